# baseline (speedup 1.0000x reference)
import functools

import jax
import jax.numpy as jnp
from jax import lax
from jax.experimental import pallas as pl
from jax.experimental.pallas import tpu as pltpu
from jax.experimental.pallas import tpu_sc as plsc

NC = 2
NS = 16
CHUNK = 128
HALF = 128
ROW_BLK = 400


def _mesh():
    return plsc.VectorSubcoreMesh(
        core_axis_name="c", subcore_axis_name="s", num_cores=NC, num_subcores=NS
    )


def _make_sc_deg(rows_tot, n_acc):
    r_deg = rows_tot // (NC * NS)
    zrows = n_acc // NS

    @functools.partial(
        pl.kernel,
        out_type=jax.ShapeDtypeStruct((NC * n_acc,), jnp.float32),
        mesh=_mesh(),
        scratch_types=[
            pltpu.MemorySpace.VMEM_SHARED((n_acc,), jnp.float32),
            pltpu.VMEM((4, 1, CHUNK), jnp.int32),
            pltpu.VMEM((CHUNK,), jnp.float32),
            pltpu.VMEM((zrows,), jnp.float32),
            pltpu.SemaphoreType.DMA((4,)),
            pltpu.SemaphoreType.DMA((2,)),
        ],
    )
    def sc_deg(dst_hbm, deg_out, deg_sh, idx_v, ones_v, zed_v, sem_i, sem_d):
        c = lax.axis_index("c")
        s = lax.axis_index("s")

        def fill_ones(i, carry):
            ones_v[pl.ds(i * 16, 16)] = jnp.ones((16,), jnp.float32)
            return carry

        lax.fori_loop(0, CHUNK // 16, fill_ones, None)

        def fill_zeros(i, carry):
            zed_v[pl.ds(i * 16, 16)] = jnp.zeros((16,), jnp.float32)
            return carry

        lax.fori_loop(0, zrows // 16, fill_zeros, None)
        pltpu.sync_copy(zed_v, deg_sh.at[pl.ds(s * zrows, zrows)])
        plsc.subcore_barrier()

        wid = c * NS + s
        row0 = wid * r_deg

        def start_idx(j):
            slot = lax.rem(j, 4)
            pltpu.async_copy(dst_hbm.at[row0 + j], idx_v.at[slot], sem_i.at[slot])

        def wait_idx(j):
            slot = lax.rem(j, 4)
            pltpu.make_async_copy(dst_hbm.at[row0], idx_v.at[slot],
                                  sem_i.at[slot]).wait()

        def start_scat(j):
            pltpu.async_copy(ones_v, deg_sh.at[idx_v.at[lax.rem(j, 4), 0]],
                             sem_d.at[lax.rem(j, 2)], add=True)

        def wait_scat(p):
            pltpu.make_async_copy(ones_v, deg_sh.at[idx_v.at[0, 0]],
                                  sem_d.at[p]).wait()

        start_idx(0)
        start_idx(1)

        def body(j, carry):
            @pl.when(j >= 2)
            def _():
                wait_scat(lax.rem(j, 2))

            @pl.when(j + 2 < r_deg)
            def _():
                start_idx(j + 2)

            wait_idx(j)
            start_scat(j)
            return carry

        lax.fori_loop(0, r_deg, body, None)
        wait_scat((r_deg - 2) % 2)
        wait_scat((r_deg - 1) % 2)
        plsc.subcore_barrier()

        @pl.when(s == 0)
        def _():
            pltpu.sync_copy(deg_sh, deg_out.at[pl.ds(c * n_acc, n_acc)])

    return sc_deg


GCH = 128
NRING = 2


def _make_sc_edge(nblk, n_acc):
    orows = n_acc // NS

    @functools.partial(
        pl.kernel,
        out_type=(
            jax.ShapeDtypeStruct((n_acc, HALF), jnp.float32),
            jax.ShapeDtypeStruct((n_acc, HALF), jnp.float32),
        ),
        mesh=_mesh(),
        scratch_types=[
            pltpu.MemorySpace.VMEM_SHARED((n_acc, HALF), jnp.float32),
            pltpu.VMEM((4, 1, GCH), jnp.int32),
            pltpu.VMEM((4, 1, GCH), jnp.int32),
            pltpu.VMEM((NRING, GCH, HALF), jnp.float32),
            pltpu.SemaphoreType.DMA((4,)),
            pltpu.SemaphoreType.DMA((NRING,)),
            pltpu.SemaphoreType.DMA((NRING,)),
        ],
    )
    def sc_edge(src_hbm, dst_hbm, g0_hbm, g1_hbm, acc0, acc1,
                acc_sh, sidx_v, didx_v, rows_v, sem_i, sem_g, sem_s):
        c = lax.axis_index("c")
        s = lax.axis_index("s")

        def fill_zeros(i, carry):
            rows_v[0, i // (HALF // 16), pl.ds((i % (HALF // 16)) * 16, 16)] = (
                jnp.zeros((16,), jnp.float32))
            return carry

        lax.fori_loop(0, GCH * (HALF // 16), fill_zeros, None)

        def zero_acc(k, carry):
            pltpu.sync_copy(rows_v.at[0],
                            acc_sh.at[pl.ds(s * orows + k * GCH, GCH)])
            return carry

        lax.fori_loop(0, orows // GCH, zero_acc, None)
        plsc.subcore_barrier()

        row0 = s * nblk

        def start_idx(k):
            slot = lax.rem(k, 4)
            pltpu.async_copy(src_hbm.at[row0 + k], sidx_v.at[slot], sem_i.at[slot])
            pltpu.async_copy(dst_hbm.at[row0 + k], didx_v.at[slot], sem_i.at[slot])

        def wait_idx(k):
            slot = lax.rem(k, 4)
            pltpu.make_async_copy(src_hbm.at[row0], sidx_v.at[slot],
                                  sem_i.at[slot]).wait()
            pltpu.make_async_copy(dst_hbm.at[row0], didx_v.at[slot],
                                  sem_i.at[slot]).wait()

        def start_gather(k, p):
            @pl.when(c == 0)
            def _():
                slot = lax.rem(k, 4)
                pltpu.async_copy(g0_hbm.at[sidx_v.at[slot, 0, pl.ds(0, 64)]],
                                 rows_v.at[p, pl.ds(0, 64)], sem_g.at[p])
                pltpu.async_copy(g0_hbm.at[sidx_v.at[slot, 0, pl.ds(64, 64)]],
                                 rows_v.at[p, pl.ds(64, 64)], sem_g.at[p])

            @pl.when(c == 1)
            def _():
                slot = lax.rem(k, 4)
                pltpu.async_copy(g1_hbm.at[sidx_v.at[slot, 0, pl.ds(0, 64)]],
                                 rows_v.at[p, pl.ds(0, 64)], sem_g.at[p])
                pltpu.async_copy(g1_hbm.at[sidx_v.at[slot, 0, pl.ds(64, 64)]],
                                 rows_v.at[p, pl.ds(64, 64)], sem_g.at[p])

        def wait_gather(p):
            pltpu.make_async_copy(g0_hbm.at[sidx_v.at[0, 0, pl.ds(0, 64)]],
                                  rows_v.at[p, pl.ds(0, 64)], sem_g.at[p]).wait()
            pltpu.make_async_copy(g0_hbm.at[sidx_v.at[0, 0, pl.ds(64, 64)]],
                                  rows_v.at[p, pl.ds(64, 64)], sem_g.at[p]).wait()

        def start_scatter(k, p):
            pltpu.async_copy(rows_v.at[p], acc_sh.at[didx_v.at[lax.rem(k, 4), 0]],
                             sem_s.at[p], add=True)

        def wait_scatter(p):
            pltpu.make_async_copy(rows_v.at[p],
                                  acc_sh.at[didx_v.at[0, 0]],
                                  sem_s.at[p]).wait()

        start_idx(0)
        start_idx(1)

        def body(k, carry):
            p = lax.rem(k, 2)
            q = 1 - p

            @pl.when(k >= 2)
            def _():
                wait_scatter(p)

            @pl.when(k + 2 < nblk)
            def _():
                start_idx(k + 2)

            wait_idx(k)
            start_gather(k, p)

            @pl.when(k >= 1)
            def _():
                wait_gather(q)
                start_scatter(k - 1, q)

            return carry

        lax.fori_loop(0, nblk, body, None)

        last = nblk - 1
        pl_last = last % 2
        wait_gather(pl_last)
        start_scatter(last, pl_last)
        wait_scatter(1 - pl_last)
        wait_scatter(pl_last)
        plsc.subcore_barrier()

        @pl.when(c == 0)
        def _():
            pltpu.sync_copy(acc_sh.at[pl.ds(s * orows, orows)],
                            acc0.at[pl.ds(s * orows, orows)])

        @pl.when(c == 1)
        def _():
            pltpu.sync_copy(acc_sh.at[pl.ds(s * orows, orows)],
                            acc1.at[pl.ds(s * orows, orows)])

    return sc_edge


def _tc_transform(x, W, deg0, deg1):
    n, d_in = x.shape
    d_h = W.shape[1]

    def body(x_ref, w_ref, d0_ref, d1_ref, g0_ref, g1_ref):
        dinv = lax.rsqrt(d0_ref[...] + d1_ref[...] + 1.0)
        h = jnp.dot(x_ref[...], w_ref[...], preferred_element_type=jnp.float32)
        g = h * dinv
        g0_ref[...] = g[:, :HALF]
        g1_ref[...] = g[:, HALF:]

    return pl.pallas_call(
        body,
        grid=(n // ROW_BLK,),
        in_specs=[
            pl.BlockSpec((ROW_BLK, d_in), lambda i: (i, 0)),
            pl.BlockSpec((d_in, d_h), lambda i: (0, 0)),
            pl.BlockSpec((ROW_BLK, 1), lambda i: (i, 0)),
            pl.BlockSpec((ROW_BLK, 1), lambda i: (i, 0)),
        ],
        out_specs=[
            pl.BlockSpec((ROW_BLK, HALF), lambda i: (i, 0)),
            pl.BlockSpec((ROW_BLK, HALF), lambda i: (i, 0)),
        ],
        out_shape=[jax.ShapeDtypeStruct((n, HALF), jnp.float32)] * 2,
    )(x, W, deg0, deg1)


def _tc_finish(acc0, acc1, g0, g1, deg0, deg1, b2, a2):
    n = g0.shape[0]
    d_h = b2.shape[1]

    def body(a0_ref, a1_ref, g0_ref, g1_ref, d0_ref, d1_ref, b_ref, al_ref, o_ref):
        dinv = lax.rsqrt(d0_ref[...] + d1_ref[...] + 1.0)
        left = a0_ref[...] + g0_ref[...]
        right = a1_ref[...] + g1_ref[...]
        pre = jnp.concatenate([left, right], axis=1) * dinv + b_ref[...]
        o_ref[...] = jnp.where(pre > 0, pre, al_ref[...] * pre)

    return pl.pallas_call(
        body,
        grid=(n // ROW_BLK,),
        in_specs=[
            pl.BlockSpec((ROW_BLK, HALF), lambda i: (i, 0)),
            pl.BlockSpec((ROW_BLK, HALF), lambda i: (i, 0)),
            pl.BlockSpec((ROW_BLK, HALF), lambda i: (i, 0)),
            pl.BlockSpec((ROW_BLK, HALF), lambda i: (i, 0)),
            pl.BlockSpec((ROW_BLK, 1), lambda i: (i, 0)),
            pl.BlockSpec((ROW_BLK, 1), lambda i: (i, 0)),
            pl.BlockSpec((1, d_h), lambda i: (0, 0)),
            pl.BlockSpec((1, d_h), lambda i: (0, 0)),
        ],
        out_specs=pl.BlockSpec((ROW_BLK, d_h), lambda i: (i, 0)),
        out_shape=jax.ShapeDtypeStruct((n, d_h), jnp.float32),
    )(acc0, acc1, g0, g1, deg0, deg1, b2, a2)


def kernel(x, edge_index, W, b, alpha):
    n = x.shape[0]
    src = edge_index[0].astype(jnp.int32)
    dst = edge_index[1].astype(jnp.int32)
    e = src.shape[0]

    quantum = NC * NS * CHUNK
    e_pad = ((e + quantum - 1) // quantum) * quantum
    pad = e_pad - e
    src_p = jnp.concatenate([src, jnp.zeros((pad,), jnp.int32)])
    dst_p = jnp.concatenate([dst, jnp.full((pad,), n, jnp.int32)])
    rows_tot = e_pad // CHUNK

    n_acc = ((n + 1 + NS * 16 - 1) // (NS * 16)) * (NS * 16)

    src3 = src_p.reshape(-1, 1, CHUNK)
    dst3 = dst_p.reshape(-1, 1, CHUNK)

    deg_flat = _make_sc_deg(rows_tot, n_acc)(dst3)
    deg0 = deg_flat[:n].reshape(n, 1)
    deg1 = deg_flat[n_acc:n_acc + n].reshape(n, 1)

    g0, g1 = _tc_transform(x, W, deg0, deg1)
    chunks_per_tile = e_pad // (NS * CHUNK)
    acc0, acc1 = _make_sc_edge(chunks_per_tile, n_acc)(src3, dst3, g0, g1)
    out = _tc_finish(acc0, acc1, g0, g1, deg0, deg1,
                     b.reshape(1, -1), alpha.reshape(1, -1))
    return out


# speedup vs baseline: 1.1116x; 1.1116x over previous
import functools

import jax
import jax.numpy as jnp
from jax import lax
from jax.experimental import pallas as pl
from jax.experimental.pallas import tpu as pltpu
from jax.experimental.pallas import tpu_sc as plsc

NC = 2
NS = 16
CHUNK = 128
HALF = 128
ROW_BLK = 400


def _mesh():
    return plsc.VectorSubcoreMesh(
        core_axis_name="c", subcore_axis_name="s", num_cores=NC, num_subcores=NS
    )


def _make_sc_deg(rows_tot, n_acc):
    r_deg = rows_tot // (NC * NS)
    zrows = n_acc // NS

    @functools.partial(
        pl.kernel,
        out_type=jax.ShapeDtypeStruct((NC * n_acc,), jnp.float32),
        mesh=_mesh(),
        scratch_types=[
            pltpu.MemorySpace.VMEM_SHARED((n_acc,), jnp.float32),
            pltpu.VMEM((4, 1, CHUNK), jnp.int32),
            pltpu.VMEM((CHUNK,), jnp.float32),
            pltpu.VMEM((zrows,), jnp.float32),
            pltpu.SemaphoreType.DMA((4,)),
            pltpu.SemaphoreType.DMA((2,)),
        ],
    )
    def sc_deg(dst_hbm, deg_out, deg_sh, idx_v, ones_v, zed_v, sem_i, sem_d):
        c = lax.axis_index("c")
        s = lax.axis_index("s")

        def fill_ones(i, carry):
            ones_v[pl.ds(i * 16, 16)] = jnp.ones((16,), jnp.float32)
            return carry

        lax.fori_loop(0, CHUNK // 16, fill_ones, None)

        def fill_zeros(i, carry):
            zed_v[pl.ds(i * 16, 16)] = jnp.zeros((16,), jnp.float32)
            return carry

        lax.fori_loop(0, zrows // 16, fill_zeros, None)
        pltpu.sync_copy(zed_v, deg_sh.at[pl.ds(s * zrows, zrows)])
        plsc.subcore_barrier()

        wid = c * NS + s
        row0 = wid * r_deg

        def start_idx(j):
            slot = lax.rem(j, 4)
            pltpu.async_copy(dst_hbm.at[row0 + j], idx_v.at[slot], sem_i.at[slot])

        def wait_idx(j):
            slot = lax.rem(j, 4)
            pltpu.make_async_copy(dst_hbm.at[row0], idx_v.at[slot],
                                  sem_i.at[slot]).wait()

        def start_scat(j):
            pltpu.async_copy(ones_v, deg_sh.at[idx_v.at[lax.rem(j, 4), 0]],
                             sem_d.at[lax.rem(j, 2)], add=True)

        def wait_scat(p):
            pltpu.make_async_copy(ones_v, deg_sh.at[idx_v.at[0, 0]],
                                  sem_d.at[p]).wait()

        start_idx(0)
        start_idx(1)

        def body(j, carry):
            @pl.when(j >= 2)
            def _():
                wait_scat(lax.rem(j, 2))

            @pl.when(j + 2 < r_deg)
            def _():
                start_idx(j + 2)

            wait_idx(j)
            start_scat(j)
            return carry

        lax.fori_loop(0, r_deg, body, None)
        wait_scat((r_deg - 2) % 2)
        wait_scat((r_deg - 1) % 2)
        plsc.subcore_barrier()

        @pl.when(s == 0)
        def _():
            pltpu.sync_copy(deg_sh, deg_out.at[pl.ds(c * n_acc, n_acc)])

    return sc_deg


GCH = 128
NRING = 2


def _make_sc_edge(nblk, n_acc):
    orows = n_acc // NS

    @functools.partial(
        pl.kernel,
        out_type=(
            jax.ShapeDtypeStruct((n_acc, HALF), jnp.float32),
            jax.ShapeDtypeStruct((n_acc, HALF), jnp.float32),
        ),
        mesh=_mesh(),
        scratch_types=[
            pltpu.MemorySpace.VMEM_SHARED((n_acc, HALF), jnp.float32),
            pltpu.VMEM((4, 1, GCH), jnp.int32),
            pltpu.VMEM((4, 1, GCH), jnp.int32),
            pltpu.VMEM((NRING, GCH, HALF), jnp.float32),
            pltpu.SemaphoreType.DMA((4,)),
            pltpu.SemaphoreType.DMA((NRING,)),
            pltpu.SemaphoreType.DMA((NRING,)),
        ],
    )
    def sc_edge(src_hbm, dst_hbm, g0_hbm, g1_hbm, acc0, acc1,
                acc_sh, sidx_v, didx_v, rows_v, sem_i, sem_g, sem_s):
        c = lax.axis_index("c")
        s = lax.axis_index("s")

        def fill_zeros(i, carry):
            rows_v[0, i // (HALF // 16), pl.ds((i % (HALF // 16)) * 16, 16)] = (
                jnp.zeros((16,), jnp.float32))
            return carry

        lax.fori_loop(0, GCH * (HALF // 16), fill_zeros, None)

        def zero_acc(k, carry):
            pltpu.sync_copy(rows_v.at[0],
                            acc_sh.at[pl.ds(s * orows + k * GCH, GCH)])
            return carry

        lax.fori_loop(0, orows // GCH, zero_acc, None)
        plsc.subcore_barrier()

        row0 = s * nblk

        def start_idx(k):
            slot = lax.rem(k, 4)
            pltpu.async_copy(src_hbm.at[row0 + k], sidx_v.at[slot], sem_i.at[slot])
            pltpu.async_copy(dst_hbm.at[row0 + k], didx_v.at[slot], sem_i.at[slot])

        def wait_idx(k):
            slot = lax.rem(k, 4)
            pltpu.make_async_copy(src_hbm.at[row0], sidx_v.at[slot],
                                  sem_i.at[slot]).wait()
            pltpu.make_async_copy(dst_hbm.at[row0], didx_v.at[slot],
                                  sem_i.at[slot]).wait()

        def start_gather(k, p):
            @pl.when(c == 0)
            def _():
                slot = lax.rem(k, 4)
                pltpu.async_copy(g0_hbm.at[sidx_v.at[slot, 0, pl.ds(0, 64)]],
                                 rows_v.at[p, pl.ds(0, 64)], sem_g.at[p])
                pltpu.async_copy(g0_hbm.at[sidx_v.at[slot, 0, pl.ds(64, 64)]],
                                 rows_v.at[p, pl.ds(64, 64)], sem_g.at[p])

            @pl.when(c == 1)
            def _():
                slot = lax.rem(k, 4)
                pltpu.async_copy(g1_hbm.at[sidx_v.at[slot, 0, pl.ds(0, 64)]],
                                 rows_v.at[p, pl.ds(0, 64)], sem_g.at[p])
                pltpu.async_copy(g1_hbm.at[sidx_v.at[slot, 0, pl.ds(64, 64)]],
                                 rows_v.at[p, pl.ds(64, 64)], sem_g.at[p])

        def wait_gather(p):
            pltpu.make_async_copy(g0_hbm.at[sidx_v.at[0, 0, pl.ds(0, 64)]],
                                  rows_v.at[p, pl.ds(0, 64)], sem_g.at[p]).wait()
            pltpu.make_async_copy(g0_hbm.at[sidx_v.at[0, 0, pl.ds(64, 64)]],
                                  rows_v.at[p, pl.ds(64, 64)], sem_g.at[p]).wait()

        def start_scatter(k, p):
            pltpu.async_copy(rows_v.at[p], acc_sh.at[didx_v.at[lax.rem(k, 4), 0]],
                             sem_s.at[p], add=True)

        def wait_scatter(p):
            pltpu.make_async_copy(rows_v.at[p],
                                  acc_sh.at[didx_v.at[0, 0]],
                                  sem_s.at[p]).wait()

        start_idx(0)
        start_idx(1)

        def body(k, carry):
            p = lax.rem(k, 2)
            q = 1 - p

            @pl.when(k >= 2)
            def _():
                wait_scatter(p)

            @pl.when(k + 2 < nblk)
            def _():
                start_idx(k + 2)

            wait_idx(k)
            start_gather(k, p)

            @pl.when(k >= 1)
            def _():
                wait_gather(q)
                start_scatter(k - 1, q)

            return carry

        lax.fori_loop(0, nblk, body, None)

        last = nblk - 1
        pl_last = last % 2
        wait_gather(pl_last)
        start_scatter(last, pl_last)
        wait_scatter(1 - pl_last)
        wait_scatter(pl_last)
        plsc.subcore_barrier()

        @pl.when(c == 0)
        def _():
            pltpu.sync_copy(acc_sh.at[pl.ds(s * orows, orows)],
                            acc0.at[pl.ds(s * orows, orows)])

        @pl.when(c == 1)
        def _():
            pltpu.sync_copy(acc_sh.at[pl.ds(s * orows, orows)],
                            acc1.at[pl.ds(s * orows, orows)])

    return sc_edge


def _tc_matmul(x, W):
    n, d_in = x.shape
    d_h = W.shape[1]

    def body(x_ref, w_ref, h0_ref, h1_ref):
        h = jnp.dot(x_ref[...], w_ref[...], preferred_element_type=jnp.float32)
        h0_ref[...] = h[:, :HALF]
        h1_ref[...] = h[:, HALF:]

    return pl.pallas_call(
        body,
        grid=(n // ROW_BLK,),
        in_specs=[
            pl.BlockSpec((ROW_BLK, d_in), lambda i: (i, 0)),
            pl.BlockSpec((d_in, d_h), lambda i: (0, 0)),
        ],
        out_specs=[
            pl.BlockSpec((ROW_BLK, HALF), lambda i: (i, 0)),
            pl.BlockSpec((ROW_BLK, HALF), lambda i: (i, 0)),
        ],
        out_shape=[jax.ShapeDtypeStruct((n, HALF), jnp.float32)] * 2,
    )(x, W)


def _tc_scale(h0, h1, deg0, deg1):
    n = h0.shape[0]

    def body(h0_ref, h1_ref, d0_ref, d1_ref, g0_ref, g1_ref):
        dinv = lax.rsqrt(d0_ref[...] + d1_ref[...] + 1.0)
        g0_ref[...] = h0_ref[...] * dinv
        g1_ref[...] = h1_ref[...] * dinv

    return pl.pallas_call(
        body,
        grid=(n // ROW_BLK,),
        in_specs=[
            pl.BlockSpec((ROW_BLK, HALF), lambda i: (i, 0)),
            pl.BlockSpec((ROW_BLK, HALF), lambda i: (i, 0)),
            pl.BlockSpec((ROW_BLK, 1), lambda i: (i, 0)),
            pl.BlockSpec((ROW_BLK, 1), lambda i: (i, 0)),
        ],
        out_specs=[
            pl.BlockSpec((ROW_BLK, HALF), lambda i: (i, 0)),
            pl.BlockSpec((ROW_BLK, HALF), lambda i: (i, 0)),
        ],
        out_shape=[jax.ShapeDtypeStruct((n, HALF), jnp.float32)] * 2,
    )(h0, h1, deg0, deg1)


def _tc_finish(acc0, acc1, g0, g1, deg0, deg1, b2, a2):
    n = g0.shape[0]
    d_h = b2.shape[1]

    def body(a0_ref, a1_ref, g0_ref, g1_ref, d0_ref, d1_ref, b_ref, al_ref, o_ref):
        dinv = lax.rsqrt(d0_ref[...] + d1_ref[...] + 1.0)
        left = a0_ref[...] + g0_ref[...]
        right = a1_ref[...] + g1_ref[...]
        pre = jnp.concatenate([left, right], axis=1) * dinv + b_ref[...]
        o_ref[...] = jnp.where(pre > 0, pre, al_ref[...] * pre)

    return pl.pallas_call(
        body,
        grid=(n // ROW_BLK,),
        in_specs=[
            pl.BlockSpec((ROW_BLK, HALF), lambda i: (i, 0)),
            pl.BlockSpec((ROW_BLK, HALF), lambda i: (i, 0)),
            pl.BlockSpec((ROW_BLK, HALF), lambda i: (i, 0)),
            pl.BlockSpec((ROW_BLK, HALF), lambda i: (i, 0)),
            pl.BlockSpec((ROW_BLK, 1), lambda i: (i, 0)),
            pl.BlockSpec((ROW_BLK, 1), lambda i: (i, 0)),
            pl.BlockSpec((1, d_h), lambda i: (0, 0)),
            pl.BlockSpec((1, d_h), lambda i: (0, 0)),
        ],
        out_specs=pl.BlockSpec((ROW_BLK, d_h), lambda i: (i, 0)),
        out_shape=jax.ShapeDtypeStruct((n, d_h), jnp.float32),
    )(acc0, acc1, g0, g1, deg0, deg1, b2, a2)


def kernel(x, edge_index, W, b, alpha):
    n = x.shape[0]
    src = edge_index[0].astype(jnp.int32)
    dst = edge_index[1].astype(jnp.int32)
    e = src.shape[0]

    quantum = NC * NS * CHUNK
    e_pad = ((e + quantum - 1) // quantum) * quantum
    pad = e_pad - e
    src_p = jnp.concatenate([src, jnp.zeros((pad,), jnp.int32)])
    dst_p = jnp.concatenate([dst, jnp.full((pad,), n, jnp.int32)])
    rows_tot = e_pad // CHUNK

    n_acc = ((n + 1 + NS * 16 - 1) // (NS * 16)) * (NS * 16)

    src3 = src_p.reshape(-1, 1, CHUNK)
    dst3 = dst_p.reshape(-1, 1, CHUNK)

    deg_flat = _make_sc_deg(rows_tot, n_acc)(dst3)
    deg0 = deg_flat[:n].reshape(n, 1)
    deg1 = deg_flat[n_acc:n_acc + n].reshape(n, 1)

    h0, h1 = _tc_matmul(x, W)
    g0, g1 = _tc_scale(h0, h1, deg0, deg1)
    chunks_per_tile = e_pad // (NS * CHUNK)
    acc0, acc1 = _make_sc_edge(chunks_per_tile, n_acc)(src3, dst3, g0, g1)
    out = _tc_finish(acc0, acc1, g0, g1, deg0, deg1,
                     b.reshape(1, -1), alpha.reshape(1, -1))
    return out
